# Initial kernel scaffold; baseline (speedup 1.0000x reference)
import jax
import jax.numpy as jnp
from jax.experimental import pallas as pl


def kernel(item_ids, table):
    raise NotImplementedError("write your pallas kernel here")


# speedup vs baseline: 1.4961x; 1.4961x over previous
import functools

import jax
import jax.numpy as jnp
from jax import lax
from jax.experimental import pallas as pl
from jax.experimental.pallas import tpu as pltpu, tpu_sc as plsc


def _build_gather(B, D, CH):
    info = plsc.get_sparse_core_info()
    NC, NS = info.num_cores, info.num_subcores
    NW = NC * NS
    b_per_w = B // NW
    n_chunks = b_per_w // CH
    mesh = plsc.VectorSubcoreMesh(core_axis_name="c", subcore_axis_name="s")

    @functools.partial(
        pl.kernel,
        mesh=mesh,
        out_type=jax.ShapeDtypeStruct((B, D), jnp.float32),
        scratch_types=[
            pltpu.VMEM((CH,), jnp.int32),
            pltpu.VMEM((CH, D), jnp.float32),
            pltpu.SemaphoreType.DMA,
        ],
        compiler_params=pltpu.CompilerParams(use_tc_tiling_on_sc=False),
    )
    def gather(table_hbm, idx_hbm, out_hbm, idx_v, rows_v, sem):
        wid = lax.axis_index("s") * NC + lax.axis_index("c")
        base = wid * b_per_w

        def body(i, carry):
            off = base + i * CH
            pltpu.sync_copy(idx_hbm.at[pl.ds(off, CH)], idx_v)
            pltpu.async_copy(table_hbm.at[idx_v], rows_v, sem).wait()
            pltpu.sync_copy(rows_v, out_hbm.at[pl.ds(off, CH)])
            return carry

        lax.fori_loop(0, n_chunks, body, 0)

    return gather


def kernel(item_ids, table):
    batch, hist = item_ids.shape
    num_rows, dim = table.shape
    idx = item_ids.reshape(-1).astype(jnp.int32)
    out = _build_gather(idx.shape[0], dim, 3200)(table, idx)
    return out.reshape(batch, hist, dim)
